# baseline (speedup 1.0000x reference)
import functools

import jax
import jax.numpy as jnp
from jax import lax
from jax.experimental import pallas as pl
from jax.experimental.pallas import tpu as pltpu
from jax.experimental.pallas import tpu_sc as plsc

N_NODES = 50000
D_FEAT = 256
G_DIM = 256
NUM_GRAPHS = 256
UNITS = 256
GROUPS = 8
EPS = 1e-3

BLK = 2000
NBLK = N_NODES // BLK
NEG = -1e30


def _pass1_body(x_ref, idx_ref, glob_ref, w0_ref, b0_ref, g0_ref, be0_ref,
                w1_ref, b1_ref, g1_ref, be1_ref, w2_ref, b2_ref,
                logits_ref, s_ref,
                cg_sc, w0a_sc, w1_sc, p_sc, b0_sc, b1_sc, s_sc):
    k = pl.program_id(0)

    @pl.when(k == 0)
    def _init():
        gs = UNITS // GROUPS
        ri = jax.lax.broadcasted_iota(jnp.int32, (UNITS, UNITS), 0) // gs
        ci = jax.lax.broadcasted_iota(jnp.int32, (UNITS, UNITS), 1) // gs
        p = (ri == ci).astype(jnp.float32) * (1.0 / gs)
        p_sc[...] = p
        cg = jnp.dot(glob_ref[...], w0_ref[D_FEAT:, :],
                     preferred_element_type=jnp.float32)
        cg_sc[...] = (cg - jnp.dot(cg, p, preferred_element_type=jnp.float32)
                      ).astype(jnp.bfloat16)
        w0a = w0_ref[:D_FEAT, :]
        w0a_sc[...] = (w0a - jnp.dot(w0a, p, preferred_element_type=jnp.float32)
                       ).astype(jnp.bfloat16)
        w1 = w1_ref[...]
        w1_sc[...] = (w1 - jnp.dot(w1, p, preferred_element_type=jnp.float32)
                      ).astype(jnp.bfloat16)
        b0_sc[...] = b0_ref[...] - jnp.dot(b0_ref[...], p,
                                           preferred_element_type=jnp.float32)
        b1_sc[...] = b1_ref[...] - jnp.dot(b1_ref[...], p,
                                           preferred_element_type=jnp.float32)
        s_sc[...] = jnp.zeros((1, NUM_GRAPHS), jnp.float32)

    x = x_ref[...].astype(jnp.bfloat16)
    idx = idx_ref[0]
    oh_b = idx == jax.lax.broadcasted_iota(jnp.int32, (BLK, NUM_GRAPHS), 1)
    oh = oh_b.astype(jnp.bfloat16)

    hc = (jnp.dot(x, w0a_sc[...], preferred_element_type=jnp.float32)
          + jnp.dot(oh, cg_sc[...], preferred_element_type=jnp.float32)
          + b0_sc[...])
    var = jnp.dot(hc * hc, p_sc[...], preferred_element_type=jnp.float32)
    h = jnp.maximum(hc * jax.lax.rsqrt(var + EPS) * g0_ref[...] + be0_ref[...],
                    0.0)
    hc = jnp.dot(h.astype(jnp.bfloat16), w1_sc[...],
                 preferred_element_type=jnp.float32) + b1_sc[...]
    var = jnp.dot(hc * hc, p_sc[...], preferred_element_type=jnp.float32)
    h = jnp.maximum(hc * jax.lax.rsqrt(var + EPS) * g1_ref[...] + be1_ref[...],
                    0.0)
    l = jnp.dot(h, w2_ref[...], preferred_element_type=jnp.float32) + b2_ref[0, 0]
    logits_ref[...] = l

    e_node = jnp.exp(l)
    s_add = jax.lax.dot_general(e_node, oh, (((0,), (0,)), ((), ())),
                                preferred_element_type=jnp.float32)
    s_new = s_sc[...] + s_add
    s_sc[...] = s_new
    s_ref[...] = s_new


_SC_NW = 32
_SC_LANES = 16
_SC_CHUNK = 1568
_SC_TAIL = N_NODES - (_SC_NW - 1) * _SC_CHUNK


def _sc_norm_body(l_hbm, idx_hbm, s_hbm, att_hbm,
                  l_v, idx_v, att_v, s_v):
    wid = lax.axis_index("s") * 2 + lax.axis_index("c")
    base = wid * _SC_CHUNK
    pltpu.sync_copy(s_hbm, s_v)

    def step(i, _):
        sl = pl.ds(i * _SC_LANES, _SC_LANES)
        iv = idx_v[sl]
        sg = plsc.load_gather(s_v, [iv])
        att_v[sl] = jnp.exp(l_v[sl]) / sg
        return 0

    def run(n):
        pltpu.sync_copy(l_hbm.at[pl.ds(base, n)], l_v.at[pl.ds(0, n)])
        pltpu.sync_copy(idx_hbm.at[pl.ds(base, n)], idx_v.at[pl.ds(0, n)])
        lax.fori_loop(0, n // _SC_LANES, step, 0)
        pltpu.sync_copy(att_v.at[pl.ds(0, n)], att_hbm.at[pl.ds(base, n)])

    @pl.when(wid < _SC_NW - 1)
    def _full():
        run(_SC_CHUNK)

    @pl.when(wid == _SC_NW - 1)
    def _tail():
        run(_SC_TAIL)


_sc_norm = pl.kernel(
    _sc_norm_body,
    out_type=jax.ShapeDtypeStruct((N_NODES,), jnp.float32),
    mesh=plsc.VectorSubcoreMesh(core_axis_name="c", subcore_axis_name="s"),
    scratch_types=[
        pltpu.VMEM((_SC_CHUNK,), jnp.float32),
        pltpu.VMEM((_SC_CHUNK,), jnp.int32),
        pltpu.VMEM((_SC_CHUNK,), jnp.float32),
        pltpu.VMEM((NUM_GRAPHS,), jnp.float32),
    ],
    compiler_params=pltpu.CompilerParams(needs_layout_passes=False),
)


@jax.jit
def kernel(inputs, graph_indices, global_features, W0, b0, gamma0, beta0,
           W1, b1, gamma1, beta1, W2, b2):
    idx3 = graph_indices.astype(jnp.int32).reshape(NBLK, BLK, 1)
    row = lambda v: v.reshape(1, -1)

    logits, s = pl.pallas_call(
        _pass1_body,
        grid=(NBLK,),
        in_specs=[
            pl.BlockSpec((BLK, D_FEAT), lambda k: (k, 0)),
            pl.BlockSpec((1, BLK, 1), lambda k: (k, 0, 0)),
            pl.BlockSpec((NUM_GRAPHS, G_DIM), lambda k: (0, 0)),
            pl.BlockSpec((D_FEAT + G_DIM, UNITS), lambda k: (0, 0)),
            pl.BlockSpec((1, UNITS), lambda k: (0, 0)),
            pl.BlockSpec((1, UNITS), lambda k: (0, 0)),
            pl.BlockSpec((1, UNITS), lambda k: (0, 0)),
            pl.BlockSpec((UNITS, UNITS), lambda k: (0, 0)),
            pl.BlockSpec((1, UNITS), lambda k: (0, 0)),
            pl.BlockSpec((1, UNITS), lambda k: (0, 0)),
            pl.BlockSpec((1, UNITS), lambda k: (0, 0)),
            pl.BlockSpec((UNITS, 1), lambda k: (0, 0)),
            pl.BlockSpec((1, 1), lambda k: (0, 0)),
        ],
        out_specs=[
            pl.BlockSpec((BLK, 1), lambda k: (k, 0)),
            pl.BlockSpec((1, NUM_GRAPHS), lambda k: (0, 0)),
        ],
        out_shape=[
            jax.ShapeDtypeStruct((N_NODES, 1), jnp.float32),
            jax.ShapeDtypeStruct((1, NUM_GRAPHS), jnp.float32),
        ],
        scratch_shapes=[
            pltpu.VMEM((G_DIM, UNITS), jnp.bfloat16),
            pltpu.VMEM((D_FEAT, UNITS), jnp.bfloat16),
            pltpu.VMEM((UNITS, UNITS), jnp.bfloat16),
            pltpu.VMEM((UNITS, UNITS), jnp.float32),
            pltpu.VMEM((1, UNITS), jnp.float32),
            pltpu.VMEM((1, UNITS), jnp.float32),
            pltpu.VMEM((1, NUM_GRAPHS), jnp.float32),
        ],
        compiler_params=pltpu.CompilerParams(
            dimension_semantics=("arbitrary",)),
    )(inputs, idx3, global_features, W0, row(b0), row(gamma0), row(beta0),
      W1, row(b1), row(gamma1), row(beta1), W2, b2.reshape(1, 1))

    att = _sc_norm(logits.reshape(N_NODES), graph_indices.astype(jnp.int32),
                   s.reshape(NUM_GRAPHS))
    return att[:, None]


# speedup vs baseline: 1.0894x; 1.0894x over previous
import functools

import jax
import jax.numpy as jnp
from jax import lax
from jax.experimental import pallas as pl
from jax.experimental.pallas import tpu as pltpu
from jax.experimental.pallas import tpu_sc as plsc

N_NODES = 50000
D_FEAT = 256
G_DIM = 256
NUM_GRAPHS = 256
UNITS = 256
GROUPS = 8
EPS = 1e-3

BLK = 5000
NBLK = N_NODES // BLK
NEG = -1e30


def _pass1_body(x_ref, idx_ref, glob_ref, w0_ref, b0_ref, g0_ref, be0_ref,
                w1_ref, b1_ref, g1_ref, be1_ref, w2_ref, b2_ref,
                logits_ref, s_ref,
                cg_sc, w0a_sc, w1_sc, p_sc, b0_sc, b1_sc, s_sc):
    k = pl.program_id(0)

    @pl.when(k == 0)
    def _init():
        gs = UNITS // GROUPS
        ri = jax.lax.broadcasted_iota(jnp.int32, (UNITS, UNITS), 0) // gs
        ci = jax.lax.broadcasted_iota(jnp.int32, (UNITS, UNITS), 1) // gs
        p = (ri == ci).astype(jnp.float32) * (1.0 / gs)
        p_sc[...] = p
        cg = jnp.dot(glob_ref[...], w0_ref[D_FEAT:, :],
                     preferred_element_type=jnp.float32)
        cg_sc[...] = cg - jnp.dot(cg, p, preferred_element_type=jnp.float32)
        w0a = w0_ref[:D_FEAT, :]
        w0a_sc[...] = w0a - jnp.dot(w0a, p, preferred_element_type=jnp.float32)
        w1 = w1_ref[...]
        w1_sc[...] = w1 - jnp.dot(w1, p, preferred_element_type=jnp.float32)
        b0_sc[...] = b0_ref[...] - jnp.dot(b0_ref[...], p,
                                           preferred_element_type=jnp.float32)
        b1_sc[...] = b1_ref[...] - jnp.dot(b1_ref[...], p,
                                           preferred_element_type=jnp.float32)
        s_sc[...] = jnp.zeros((1, NUM_GRAPHS), jnp.float32)

    x = x_ref[...]
    idx = idx_ref[0]
    oh_b = idx == jax.lax.broadcasted_iota(jnp.int32, (BLK, NUM_GRAPHS), 1)
    oh = oh_b.astype(jnp.float32)

    hc = (jnp.dot(x, w0a_sc[...], preferred_element_type=jnp.float32)
          + jnp.dot(oh, cg_sc[...], preferred_element_type=jnp.float32)
          + b0_sc[...])
    var = jnp.dot(hc * hc, p_sc[...], preferred_element_type=jnp.float32)
    h = jnp.maximum(hc * jax.lax.rsqrt(var + EPS) * g0_ref[...] + be0_ref[...],
                    0.0)
    hc = jnp.dot(h, w1_sc[...], preferred_element_type=jnp.float32) + b1_sc[...]
    var = jnp.dot(hc * hc, p_sc[...], preferred_element_type=jnp.float32)
    h = jnp.maximum(hc * jax.lax.rsqrt(var + EPS) * g1_ref[...] + be1_ref[...],
                    0.0)
    l = jnp.dot(h, w2_ref[...], preferred_element_type=jnp.float32) + b2_ref[0, 0]
    logits_ref[...] = l

    e_node = jnp.exp(l)
    s_add = jax.lax.dot_general(e_node, oh, (((0,), (0,)), ((), ())),
                                preferred_element_type=jnp.float32)
    s_new = s_sc[...] + s_add
    s_sc[...] = s_new
    s_ref[...] = s_new


_SC_NW = 32
_SC_LANES = 16
_SC_CHUNK = 1568
_SC_TAIL = N_NODES - (_SC_NW - 1) * _SC_CHUNK


def _sc_norm_body(l_hbm, idx_hbm, s_hbm, att_hbm,
                  l_v, idx_v, att_v, s_v):
    wid = lax.axis_index("s") * 2 + lax.axis_index("c")
    base = wid * _SC_CHUNK
    pltpu.sync_copy(s_hbm, s_v)

    def step(i, _):
        sl = pl.ds(i * _SC_LANES, _SC_LANES)
        iv = idx_v[sl]
        sg = plsc.load_gather(s_v, [iv])
        att_v[sl] = jnp.exp(l_v[sl]) / sg
        return 0

    def run(n):
        pltpu.sync_copy(l_hbm.at[pl.ds(base, n)], l_v.at[pl.ds(0, n)])
        pltpu.sync_copy(idx_hbm.at[pl.ds(base, n)], idx_v.at[pl.ds(0, n)])
        lax.fori_loop(0, n // _SC_LANES, step, 0)
        pltpu.sync_copy(att_v.at[pl.ds(0, n)], att_hbm.at[pl.ds(base, n)])

    @pl.when(wid < _SC_NW - 1)
    def _full():
        run(_SC_CHUNK)

    @pl.when(wid == _SC_NW - 1)
    def _tail():
        run(_SC_TAIL)


_sc_norm = pl.kernel(
    _sc_norm_body,
    out_type=jax.ShapeDtypeStruct((N_NODES,), jnp.float32),
    mesh=plsc.VectorSubcoreMesh(core_axis_name="c", subcore_axis_name="s"),
    scratch_types=[
        pltpu.VMEM((_SC_CHUNK,), jnp.float32),
        pltpu.VMEM((_SC_CHUNK,), jnp.int32),
        pltpu.VMEM((_SC_CHUNK,), jnp.float32),
        pltpu.VMEM((NUM_GRAPHS,), jnp.float32),
    ],
    compiler_params=pltpu.CompilerParams(needs_layout_passes=False),
)


@jax.jit
def kernel(inputs, graph_indices, global_features, W0, b0, gamma0, beta0,
           W1, b1, gamma1, beta1, W2, b2):
    idx3 = graph_indices.astype(jnp.int32).reshape(NBLK, BLK, 1)
    row = lambda v: v.reshape(1, -1)

    logits, s = pl.pallas_call(
        _pass1_body,
        grid=(NBLK,),
        in_specs=[
            pl.BlockSpec((BLK, D_FEAT), lambda k: (k, 0)),
            pl.BlockSpec((1, BLK, 1), lambda k: (k, 0, 0)),
            pl.BlockSpec((NUM_GRAPHS, G_DIM), lambda k: (0, 0)),
            pl.BlockSpec((D_FEAT + G_DIM, UNITS), lambda k: (0, 0)),
            pl.BlockSpec((1, UNITS), lambda k: (0, 0)),
            pl.BlockSpec((1, UNITS), lambda k: (0, 0)),
            pl.BlockSpec((1, UNITS), lambda k: (0, 0)),
            pl.BlockSpec((UNITS, UNITS), lambda k: (0, 0)),
            pl.BlockSpec((1, UNITS), lambda k: (0, 0)),
            pl.BlockSpec((1, UNITS), lambda k: (0, 0)),
            pl.BlockSpec((1, UNITS), lambda k: (0, 0)),
            pl.BlockSpec((UNITS, 1), lambda k: (0, 0)),
            pl.BlockSpec((1, 1), lambda k: (0, 0)),
        ],
        out_specs=[
            pl.BlockSpec((BLK, 1), lambda k: (k, 0)),
            pl.BlockSpec((1, NUM_GRAPHS), lambda k: (0, 0)),
        ],
        out_shape=[
            jax.ShapeDtypeStruct((N_NODES, 1), jnp.float32),
            jax.ShapeDtypeStruct((1, NUM_GRAPHS), jnp.float32),
        ],
        scratch_shapes=[
            pltpu.VMEM((G_DIM, UNITS), jnp.float32),
            pltpu.VMEM((D_FEAT, UNITS), jnp.float32),
            pltpu.VMEM((UNITS, UNITS), jnp.float32),
            pltpu.VMEM((UNITS, UNITS), jnp.float32),
            pltpu.VMEM((1, UNITS), jnp.float32),
            pltpu.VMEM((1, UNITS), jnp.float32),
            pltpu.VMEM((1, NUM_GRAPHS), jnp.float32),
        ],
        compiler_params=pltpu.CompilerParams(
            dimension_semantics=("arbitrary",)),
    )(inputs, idx3, global_features, W0, row(b0), row(gamma0), row(beta0),
      W1, row(b1), row(gamma1), row(beta1), W2, b2.reshape(1, 1))

    att = _sc_norm(logits.reshape(N_NODES), graph_indices.astype(jnp.int32),
                   s.reshape(NUM_GRAPHS))
    return att[:, None]


# speedup vs baseline: 1.1084x; 1.0175x over previous
import functools

import jax
import jax.numpy as jnp
from jax import lax
from jax.experimental import pallas as pl
from jax.experimental.pallas import tpu as pltpu
from jax.experimental.pallas import tpu_sc as plsc

N_NODES = 50000
D_FEAT = 256
G_DIM = 256
NUM_GRAPHS = 256
UNITS = 256
GROUPS = 8
EPS = 1e-3

BLK = 5000
NBLK = N_NODES // BLK
NEG = -1e30


def _pass1_body(x_ref, idx_ref, glob_ref, w0_ref, b0_ref, g0_ref, be0_ref,
                w1_ref, b1_ref, g1_ref, be1_ref, w2_ref, b2_ref,
                logits_ref, s_ref,
                cg_sc, w0a_sc, w1_sc, p_sc, b0_sc, b1_sc, s_sc):
    k = pl.program_id(0)

    @pl.when(k == 0)
    def _init():
        gs = UNITS // GROUPS
        ri = jax.lax.broadcasted_iota(jnp.int32, (UNITS, UNITS), 0) // gs
        ci = jax.lax.broadcasted_iota(jnp.int32, (UNITS, UNITS), 1) // gs
        p = (ri == ci).astype(jnp.float32) * (1.0 / gs)
        p_sc[...] = p
        cg = jnp.dot(glob_ref[...], w0_ref[D_FEAT:, :],
                     preferred_element_type=jnp.float32)
        cg_sc[...] = (cg - jnp.dot(cg, p, preferred_element_type=jnp.float32)
                      ).astype(jnp.bfloat16)
        w0a = w0_ref[:D_FEAT, :]
        w0a_sc[...] = w0a - jnp.dot(w0a, p, preferred_element_type=jnp.float32)
        w1 = w1_ref[...]
        w1_sc[...] = w1 - jnp.dot(w1, p, preferred_element_type=jnp.float32)
        b0_sc[...] = b0_ref[...] - jnp.dot(b0_ref[...], p,
                                           preferred_element_type=jnp.float32)
        b1_sc[...] = b1_ref[...] - jnp.dot(b1_ref[...], p,
                                           preferred_element_type=jnp.float32)
        s_sc[...] = jnp.zeros((1, NUM_GRAPHS), jnp.float32)

    x = x_ref[...]
    idx = idx_ref[0]
    oh_b = idx == jax.lax.broadcasted_iota(jnp.int32, (BLK, NUM_GRAPHS), 1)
    oh = oh_b.astype(jnp.bfloat16)

    hc = (jnp.dot(x, w0a_sc[...], preferred_element_type=jnp.float32)
          + jnp.dot(oh, cg_sc[...], preferred_element_type=jnp.float32)
          + b0_sc[...])
    var = jnp.dot(hc * hc, p_sc[...], preferred_element_type=jnp.float32)
    h = jnp.maximum(hc * jax.lax.rsqrt(var + EPS) * g0_ref[...] + be0_ref[...],
                    0.0)
    hc = jnp.dot(h, w1_sc[...], preferred_element_type=jnp.float32) + b1_sc[...]
    var = jnp.dot(hc * hc, p_sc[...], preferred_element_type=jnp.float32)
    h = jnp.maximum(hc * jax.lax.rsqrt(var + EPS) * g1_ref[...] + be1_ref[...],
                    0.0)
    l = jnp.dot(h, w2_ref[...], preferred_element_type=jnp.float32) + b2_ref[0, 0]
    logits_ref[...] = l

    e_node = jnp.exp(l)
    s_add = jax.lax.dot_general(e_node.astype(jnp.bfloat16), oh,
                                (((0,), (0,)), ((), ())),
                                preferred_element_type=jnp.float32)
    s_new = s_sc[...] + s_add
    s_sc[...] = s_new
    s_ref[...] = s_new


_SC_NW = 32
_SC_LANES = 16
_SC_CHUNK = 1568
_SC_TAIL = N_NODES - (_SC_NW - 1) * _SC_CHUNK


def _sc_norm_body(l_hbm, idx_hbm, s_hbm, att_hbm,
                  l_v, idx_v, att_v, s_v):
    wid = lax.axis_index("s") * 2 + lax.axis_index("c")
    base = wid * _SC_CHUNK
    pltpu.sync_copy(s_hbm, s_v)

    def step(i, _):
        sl = pl.ds(i * _SC_LANES, _SC_LANES)
        iv = idx_v[sl]
        sg = plsc.load_gather(s_v, [iv])
        att_v[sl] = jnp.exp(l_v[sl]) / sg
        return 0

    def run(n):
        pltpu.sync_copy(l_hbm.at[pl.ds(base, n)], l_v.at[pl.ds(0, n)])
        pltpu.sync_copy(idx_hbm.at[pl.ds(base, n)], idx_v.at[pl.ds(0, n)])
        lax.fori_loop(0, n // _SC_LANES, step, 0)
        pltpu.sync_copy(att_v.at[pl.ds(0, n)], att_hbm.at[pl.ds(base, n)])

    @pl.when(wid < _SC_NW - 1)
    def _full():
        run(_SC_CHUNK)

    @pl.when(wid == _SC_NW - 1)
    def _tail():
        run(_SC_TAIL)


_sc_norm = pl.kernel(
    _sc_norm_body,
    out_type=jax.ShapeDtypeStruct((N_NODES,), jnp.float32),
    mesh=plsc.VectorSubcoreMesh(core_axis_name="c", subcore_axis_name="s"),
    scratch_types=[
        pltpu.VMEM((_SC_CHUNK,), jnp.float32),
        pltpu.VMEM((_SC_CHUNK,), jnp.int32),
        pltpu.VMEM((_SC_CHUNK,), jnp.float32),
        pltpu.VMEM((NUM_GRAPHS,), jnp.float32),
    ],
    compiler_params=pltpu.CompilerParams(needs_layout_passes=False),
)


@jax.jit
def kernel(inputs, graph_indices, global_features, W0, b0, gamma0, beta0,
           W1, b1, gamma1, beta1, W2, b2):
    idx3 = graph_indices.astype(jnp.int32).reshape(NBLK, BLK, 1)
    row = lambda v: v.reshape(1, -1)

    logits, s = pl.pallas_call(
        _pass1_body,
        grid=(NBLK,),
        in_specs=[
            pl.BlockSpec((BLK, D_FEAT), lambda k: (k, 0)),
            pl.BlockSpec((1, BLK, 1), lambda k: (k, 0, 0)),
            pl.BlockSpec((NUM_GRAPHS, G_DIM), lambda k: (0, 0)),
            pl.BlockSpec((D_FEAT + G_DIM, UNITS), lambda k: (0, 0)),
            pl.BlockSpec((1, UNITS), lambda k: (0, 0)),
            pl.BlockSpec((1, UNITS), lambda k: (0, 0)),
            pl.BlockSpec((1, UNITS), lambda k: (0, 0)),
            pl.BlockSpec((UNITS, UNITS), lambda k: (0, 0)),
            pl.BlockSpec((1, UNITS), lambda k: (0, 0)),
            pl.BlockSpec((1, UNITS), lambda k: (0, 0)),
            pl.BlockSpec((1, UNITS), lambda k: (0, 0)),
            pl.BlockSpec((UNITS, 1), lambda k: (0, 0)),
            pl.BlockSpec((1, 1), lambda k: (0, 0)),
        ],
        out_specs=[
            pl.BlockSpec((BLK, 1), lambda k: (k, 0)),
            pl.BlockSpec((1, NUM_GRAPHS), lambda k: (0, 0)),
        ],
        out_shape=[
            jax.ShapeDtypeStruct((N_NODES, 1), jnp.float32),
            jax.ShapeDtypeStruct((1, NUM_GRAPHS), jnp.float32),
        ],
        scratch_shapes=[
            pltpu.VMEM((G_DIM, UNITS), jnp.bfloat16),
            pltpu.VMEM((D_FEAT, UNITS), jnp.float32),
            pltpu.VMEM((UNITS, UNITS), jnp.float32),
            pltpu.VMEM((UNITS, UNITS), jnp.float32),
            pltpu.VMEM((1, UNITS), jnp.float32),
            pltpu.VMEM((1, UNITS), jnp.float32),
            pltpu.VMEM((1, NUM_GRAPHS), jnp.float32),
        ],
        compiler_params=pltpu.CompilerParams(
            dimension_semantics=("arbitrary",)),
    )(inputs, idx3, global_features, W0, row(b0), row(gamma0), row(beta0),
      W1, row(b1), row(gamma1), row(beta1), W2, b2.reshape(1, 1))

    att = _sc_norm(logits.reshape(N_NODES), graph_indices.astype(jnp.int32),
                   s.reshape(NUM_GRAPHS))
    return att[:, None]
